# baseline (speedup 1.0000x reference)
import functools

import jax
import jax.numpy as jnp
from jax.experimental import pallas as pl
from jax.experimental.pallas import tpu as pltpu


def _prologue_kernel(x_ref, wt_ref, r1_ref, r2_ref, wh_ref, s1_ref, s2_ref):
    wh = jnp.dot(x_ref[...], wt_ref[...], preferred_element_type=jnp.float32)
    wh_ref[...] = wh
    s1_ref[...] = jnp.dot(wh, r1_ref[...], preferred_element_type=jnp.float32)
    s2_ref[...] = jnp.dot(wh, r2_ref[...], preferred_element_type=jnp.float32)


def _attn_kernel(a_ref, s1_ref, s2_ref, wh_ref, out_ref, acc_ref, den_ref):
    j = pl.program_id(1)
    nj = pl.num_programs(1)

    @pl.when(j == 0)
    def _():
        acc_ref[...] = jnp.zeros_like(acc_ref)
        den_ref[...] = jnp.zeros_like(den_ref)

    a = a_ref[...]
    e = s1_ref[...] + s2_ref[...]
    e = jnp.where(e >= 0.0, e, 0.2 * e)
    p = jnp.exp(e)
    mask = a != 0.0
    attn = jnp.where(mask, p, 1.0)
    den_ref[...] += jnp.sum(jnp.where(mask, p, 0.0), axis=1, keepdims=True)
    acc_ref[...] += jnp.dot(attn, wh_ref[...], preferred_element_type=jnp.float32)

    @pl.when(j == nj - 1)
    def _():
        x = acc_ref[...] / den_ref[...]
        out_ref[...] = 0.5 * x * (1.0 + jax.lax.erf(x * 0.7071067811865476))


@jax.jit
def kernel(X, A, W, r):
    n, d_in = X.shape
    d_out = W.shape[0]

    bm = 256
    bk = 512

    wh, s1, s2 = pl.pallas_call(
        _prologue_kernel,
        grid=(n // bm,),
        in_specs=[
            pl.BlockSpec((bm, d_in), lambda i: (i, 0)),
            pl.BlockSpec((d_in, d_out), lambda i: (0, 0)),
            pl.BlockSpec((d_out, 1), lambda i: (0, 0)),
            pl.BlockSpec((d_out, 1), lambda i: (0, 0)),
        ],
        out_specs=[
            pl.BlockSpec((bm, d_out), lambda i: (i, 0)),
            pl.BlockSpec((bm, 1), lambda i: (i, 0)),
            pl.BlockSpec((bm, 1), lambda i: (i, 0)),
        ],
        out_shape=[
            jax.ShapeDtypeStruct((n, d_out), jnp.float32),
            jax.ShapeDtypeStruct((n, 1), jnp.float32),
            jax.ShapeDtypeStruct((n, 1), jnp.float32),
        ],
    )(X, W.T, r[:d_out], r[d_out:])

    s2_row = s2.reshape(1, n)

    out = pl.pallas_call(
        _attn_kernel,
        grid=(n // bm, n // bk),
        in_specs=[
            pl.BlockSpec((bm, bk), lambda i, j: (i, j)),
            pl.BlockSpec((bm, 1), lambda i, j: (i, 0)),
            pl.BlockSpec((1, bk), lambda i, j: (0, j)),
            pl.BlockSpec((bk, d_out), lambda i, j: (j, 0)),
        ],
        out_specs=pl.BlockSpec((bm, d_out), lambda i, j: (i, 0)),
        out_shape=jax.ShapeDtypeStruct((n, d_out), jnp.float32),
        scratch_shapes=[
            pltpu.VMEM((bm, d_out), jnp.float32),
            pltpu.VMEM((bm, 1), jnp.float32),
        ],
        compiler_params=pltpu.CompilerParams(
            dimension_semantics=("parallel", "arbitrary"),
        ),
    )(A, s1, s2_row, wh)

    return out


# speedup vs baseline: 1.7450x; 1.7450x over previous
import jax
import jax.numpy as jnp
from jax.experimental import pallas as pl
from jax.experimental.pallas import tpu as pltpu


def _prologue_kernel(x_ref, wt_ref, r1_ref, r2_ref,
                     wh_ref, e1_ref, f1_ref, e2_ref, f2_ref):
    wh = jnp.dot(x_ref[...], wt_ref[...], preferred_element_type=jnp.float32)
    wh_ref[...] = wh
    s1 = jnp.dot(wh, r1_ref[...], preferred_element_type=jnp.float32)
    s2 = jnp.dot(wh, r2_ref[...], preferred_element_type=jnp.float32)
    e1_ref[...] = jnp.exp(s1)
    f1_ref[...] = jnp.exp(0.2 * s1)
    e2_ref[...] = jnp.exp(s2)
    f2_ref[...] = jnp.exp(0.2 * s2)


def _attn_kernel(a_ref, e1_ref, f1_ref, e2_ref, f2_ref, wh_ref,
                 out_ref, acc_ref, den_ref):
    j = pl.program_id(1)
    nj = pl.num_programs(1)

    @pl.when(j == 0)
    def _():
        acc_ref[...] = jnp.zeros_like(acc_ref)
        den_ref[...] = jnp.zeros_like(den_ref)

    a = a_ref[...]
    p = jnp.maximum(e1_ref[...] * e2_ref[...], f1_ref[...] * f2_ref[...])
    attn = jnp.where(a != 0.0, p, 1.0)
    den_ref[...] += jnp.sum(attn * a, axis=1, keepdims=True)
    acc_ref[...] += jnp.dot(attn, wh_ref[...], preferred_element_type=jnp.float32)

    @pl.when(j == nj - 1)
    def _():
        x = acc_ref[...] / den_ref[...]
        out_ref[...] = 0.5 * x * (1.0 + jax.lax.erf(x * 0.7071067811865476))


@jax.jit
def kernel(X, A, W, r):
    n, d_in = X.shape
    d_out = W.shape[0]

    bm = 256
    bk = 2048

    vec = jax.ShapeDtypeStruct((n, 1), jnp.float32)
    wh, e1, f1, e2, f2 = pl.pallas_call(
        _prologue_kernel,
        grid=(n // bm,),
        in_specs=[
            pl.BlockSpec((bm, d_in), lambda i: (i, 0)),
            pl.BlockSpec((d_in, d_out), lambda i: (0, 0)),
            pl.BlockSpec((d_out, 1), lambda i: (0, 0)),
            pl.BlockSpec((d_out, 1), lambda i: (0, 0)),
        ],
        out_specs=[
            pl.BlockSpec((bm, d_out), lambda i: (i, 0)),
            pl.BlockSpec((bm, 1), lambda i: (i, 0)),
            pl.BlockSpec((bm, 1), lambda i: (i, 0)),
            pl.BlockSpec((bm, 1), lambda i: (i, 0)),
            pl.BlockSpec((bm, 1), lambda i: (i, 0)),
        ],
        out_shape=[
            jax.ShapeDtypeStruct((n, d_out), jnp.float32),
            vec, vec, vec, vec,
        ],
    )(X, W.T, r[:d_out], r[d_out:])

    e2r = e2.reshape(1, n)
    f2r = f2.reshape(1, n)

    out = pl.pallas_call(
        _attn_kernel,
        grid=(n // bm, n // bk),
        in_specs=[
            pl.BlockSpec((bm, bk), lambda i, j: (i, j)),
            pl.BlockSpec((bm, 1), lambda i, j: (i, 0)),
            pl.BlockSpec((bm, 1), lambda i, j: (i, 0)),
            pl.BlockSpec((1, bk), lambda i, j: (0, j)),
            pl.BlockSpec((1, bk), lambda i, j: (0, j)),
            pl.BlockSpec((bk, d_out), lambda i, j: (j, 0)),
        ],
        out_specs=pl.BlockSpec((bm, d_out), lambda i, j: (i, 0)),
        out_shape=jax.ShapeDtypeStruct((n, d_out), jnp.float32),
        scratch_shapes=[
            pltpu.VMEM((bm, d_out), jnp.float32),
            pltpu.VMEM((bm, 1), jnp.float32),
        ],
        compiler_params=pltpu.CompilerParams(
            dimension_semantics=("parallel", "arbitrary"),
        ),
    )(A, e1, f1, e2r, f2r, wh)

    return out


# speedup vs baseline: 2.5212x; 1.4449x over previous
import jax
import jax.numpy as jnp
from jax.experimental import pallas as pl
from jax.experimental.pallas import tpu as pltpu


def _prologue_kernel(x_ref, wt_ref, r1_ref, r2_ref,
                     wh_ref, e1_ref, f1_ref, e2_ref, f2_ref):
    wh = jnp.dot(x_ref[...], wt_ref[...], preferred_element_type=jnp.float32)
    wh_ref[...] = wh
    s1 = jnp.dot(wh, r1_ref[...], preferred_element_type=jnp.float32)
    s2 = jnp.dot(wh, r2_ref[...], preferred_element_type=jnp.float32)
    e1_ref[...] = jnp.exp(s1)
    f1_ref[...] = jnp.exp(0.2 * s1)
    e2_ref[...] = jnp.exp(s2)
    f2_ref[...] = jnp.exp(0.2 * s2)


def _attn_kernel(a_ref, e1_ref, f1_ref, e2_ref, f2_ref, wh_ref, out_ref):
    a = a_ref[...]
    p = jnp.maximum(e1_ref[...] * e2_ref[...], f1_ref[...] * f2_ref[...])
    attn = jnp.where(a != 0.0, p, 1.0)
    den = jnp.sum(attn * a, axis=1, keepdims=True)
    acc = jnp.dot(attn, wh_ref[...], preferred_element_type=jnp.float32)
    x = acc / den
    out_ref[...] = 0.5 * x * (1.0 + jax.lax.erf(x * 0.7071067811865476))


@jax.jit
def kernel(X, A, W, r):
    n, d_in = X.shape
    d_out = W.shape[0]

    bm = 256

    vec = jax.ShapeDtypeStruct((n, 1), jnp.float32)
    wh, e1, f1, e2, f2 = pl.pallas_call(
        _prologue_kernel,
        grid=(1,),
        in_specs=[
            pl.BlockSpec((n, d_in), lambda i: (0, 0)),
            pl.BlockSpec((d_in, d_out), lambda i: (0, 0)),
            pl.BlockSpec((d_out, 1), lambda i: (0, 0)),
            pl.BlockSpec((d_out, 1), lambda i: (0, 0)),
        ],
        out_specs=[
            pl.BlockSpec((n, d_out), lambda i: (0, 0)),
            pl.BlockSpec((n, 1), lambda i: (0, 0)),
            pl.BlockSpec((n, 1), lambda i: (0, 0)),
            pl.BlockSpec((n, 1), lambda i: (0, 0)),
            pl.BlockSpec((n, 1), lambda i: (0, 0)),
        ],
        out_shape=[
            jax.ShapeDtypeStruct((n, d_out), jnp.float32),
            vec, vec, vec, vec,
        ],
    )(X, W.T, r[:d_out], r[d_out:])

    e2r = e2.reshape(1, n)
    f2r = f2.reshape(1, n)

    out = pl.pallas_call(
        _attn_kernel,
        grid=(n // bm,),
        in_specs=[
            pl.BlockSpec((bm, n), lambda i: (i, 0)),
            pl.BlockSpec((bm, 1), lambda i: (i, 0)),
            pl.BlockSpec((bm, 1), lambda i: (i, 0)),
            pl.BlockSpec((1, n), lambda i: (0, 0)),
            pl.BlockSpec((1, n), lambda i: (0, 0)),
            pl.BlockSpec((n, d_out), lambda i: (0, 0)),
        ],
        out_specs=pl.BlockSpec((bm, d_out), lambda i: (i, 0)),
        out_shape=jax.ShapeDtypeStruct((n, d_out), jnp.float32),
        compiler_params=pltpu.CompilerParams(
            dimension_semantics=("arbitrary",),
        ),
    )(A, e1, f1, e2r, f2r, wh)

    return out


# speedup vs baseline: 2.7485x; 1.0902x over previous
import jax
import jax.numpy as jnp
from jax.experimental import pallas as pl
from jax.experimental.pallas import tpu as pltpu


def _prologue_kernel(x_ref, wt_ref, r1_ref, r2_ref,
                     wh_ref, e1_ref, f1_ref, e2_ref, f2_ref):
    wh = jnp.dot(x_ref[...], wt_ref[...], preferred_element_type=jnp.float32)
    wh_ref[...] = wh
    s1 = jnp.dot(wh, r1_ref[...], preferred_element_type=jnp.float32)
    s2 = jnp.dot(wh, r2_ref[...], preferred_element_type=jnp.float32)
    e1_ref[...] = jnp.exp(s1)
    f1_ref[...] = jnp.exp(0.2 * s1)
    e2_ref[...] = jnp.exp(s2)
    f2_ref[...] = jnp.exp(0.2 * s2)


def _attn_kernel(a_ref, e1_ref, f1_ref, e2_ref, f2_ref, wh_ref, out_ref):
    a = a_ref[...]
    p = jnp.maximum(e1_ref[...] * e2_ref[...], f1_ref[...] * f2_ref[...])
    attn = jnp.where(a != 0.0, p, 1.0)
    den = jnp.sum(attn * a, axis=1, keepdims=True)
    acc = jnp.dot(attn, wh_ref[...], preferred_element_type=jnp.float32)
    x = acc / den
    out_ref[...] = 0.5 * x * (1.0 + jax.lax.erf(x * 0.7071067811865476))


@jax.jit
def kernel(X, A, W, r):
    n, d_in = X.shape
    d_out = W.shape[0]

    bm = 512

    vec = jax.ShapeDtypeStruct((n, 1), jnp.float32)
    wh, e1, f1, e2, f2 = pl.pallas_call(
        _prologue_kernel,
        grid=(1,),
        in_specs=[
            pl.BlockSpec((n, d_in), lambda i: (0, 0)),
            pl.BlockSpec((d_in, d_out), lambda i: (0, 0)),
            pl.BlockSpec((d_out, 1), lambda i: (0, 0)),
            pl.BlockSpec((d_out, 1), lambda i: (0, 0)),
        ],
        out_specs=[
            pl.BlockSpec((n, d_out), lambda i: (0, 0)),
            pl.BlockSpec((n, 1), lambda i: (0, 0)),
            pl.BlockSpec((n, 1), lambda i: (0, 0)),
            pl.BlockSpec((n, 1), lambda i: (0, 0)),
            pl.BlockSpec((n, 1), lambda i: (0, 0)),
        ],
        out_shape=[
            jax.ShapeDtypeStruct((n, d_out), jnp.float32),
            vec, vec, vec, vec,
        ],
    )(X, W.T, r[:d_out], r[d_out:])

    e2r = e2.reshape(1, n)
    f2r = f2.reshape(1, n)

    out = pl.pallas_call(
        _attn_kernel,
        grid=(n // bm,),
        in_specs=[
            pl.BlockSpec((bm, n), lambda i: (i, 0)),
            pl.BlockSpec((bm, 1), lambda i: (i, 0)),
            pl.BlockSpec((bm, 1), lambda i: (i, 0)),
            pl.BlockSpec((1, n), lambda i: (0, 0)),
            pl.BlockSpec((1, n), lambda i: (0, 0)),
            pl.BlockSpec((n, d_out), lambda i: (0, 0)),
        ],
        out_specs=pl.BlockSpec((bm, d_out), lambda i: (i, 0)),
        out_shape=jax.ShapeDtypeStruct((n, d_out), jnp.float32),
        compiler_params=pltpu.CompilerParams(
            dimension_semantics=("arbitrary",),
        ),
    )(A, e1, f1, e2r, f2r, wh)

    return out
